# Initial kernel scaffold; baseline (speedup 1.0000x reference)
import jax
import jax.numpy as jnp
from jax.experimental import pallas as pl


def kernel(x, edge_index, edge_weight, W1, b1, Wc0, bc0, Wc1, bc1, W2, b2):
    raise NotImplementedError("write your pallas kernel here")


# speedup vs baseline: 6.2219x; 6.2219x over previous
import functools

import jax
import jax.numpy as jnp
from jax import lax
from jax.experimental import pallas as pl
from jax.experimental.pallas import tpu as pltpu
from jax.experimental.pallas import tpu_sc as plsc

N = 10000
E = 320000
DIN = 256
H = 128
C = 18
K = 38

NC = 2
NS = 16
NW = NC * NS
RPT = 80
ER = NW * RPT
EP = ER * H
NPAD = N + 16
NG = N // 16
GPS = NG // NS


def _zero_shared(sh_ref, ztile, s):
    def zbody(g, carry):
        gid = s + NS * g
        pltpu.sync_copy(ztile, sh_ref.at[pl.ds(gid * 16, 16)])
        return carry
    lax.fori_loop(0, GPS, zbody, 0)

    @pl.when(s == 0)
    def _():
        pltpu.sync_copy(ztile, sh_ref.at[pl.ds(GPS * NS * 16, 16)])


def _copy_out_shared(sh_ref, out_ref, c, s, width):
    def obody(g, carry):
        gid = s + NS * g
        pltpu.sync_copy(sh_ref.at[pl.ds(gid * 16, 16)],
                        out_ref.at[pl.ds(c * N + gid * 16, 16)])
        return carry
    lax.fori_loop(0, GPS, obody, 0)

    @pl.when(s == 0)
    def _():
        base = GPS * NS * 16
        pltpu.sync_copy(sh_ref.at[pl.ds(base, 16)],
                        out_ref.at[pl.ds(c * N + base, 16)])


def _sc_degree_body(ei_hbm, out_hbm, dstv, ones_v, ztile, deg_sh):
    c = lax.axis_index("c")
    s = lax.axis_index("s")
    w = c * NS + s
    for i in range(H):
        ones_v[i, :] = jnp.ones((16,), jnp.float32)
    for i in range(16):
        ztile[i, :] = jnp.zeros((16,), jnp.float32)
    _zero_shared(deg_sh, ztile, s)
    plsc.subcore_barrier()

    pltpu.sync_copy(ei_hbm.at[1, pl.ds(w * RPT, RPT)], dstv)

    def ebody(i, carry):
        pltpu.sync_copy(ones_v, deg_sh.at[dstv.at[i]], add=True)
        return carry
    lax.fori_loop(0, RPT, ebody, 0)

    plsc.subcore_barrier()
    _copy_out_shared(deg_sh, out_hbm, c, s, 16)


def _sc_edge_body(ei_hbm, y_hbm, out_hbm, srcv, dstv, rows, ztile, sem, acc_sh):
    c = lax.axis_index("c")
    s = lax.axis_index("s")
    w = c * NS + s
    for i in range(16):
        for j in range(H // 16):
            ztile[i, pl.ds(j * 16, 16)] = jnp.zeros((16,), jnp.float32)
    _zero_shared(acc_sh, ztile, s)
    plsc.subcore_barrier()

    pltpu.sync_copy(ei_hbm.at[0, pl.ds(w * RPT, RPT)], srcv)
    pltpu.sync_copy(ei_hbm.at[1, pl.ds(w * RPT, RPT)], dstv)

    def ebody(i, carry):
        pltpu.async_copy(y_hbm.at[srcv.at[i]], rows, sem).wait()
        pltpu.sync_copy(rows, acc_sh.at[dstv.at[i]], add=True)
        return carry
    lax.fori_loop(0, RPT, ebody, 0)

    plsc.subcore_barrier()
    _copy_out_shared(acc_sh, out_hbm, c, s, H)


@functools.cache
def _build_sc_kernels():
    mesh = plsc.VectorSubcoreMesh(core_axis_name="c", subcore_axis_name="s",
                                  num_cores=NC, num_subcores=NS)
    deg_k = pl.kernel(
        _sc_degree_body,
        out_type=jax.ShapeDtypeStruct((NC * N, 16), jnp.float32),
        mesh=mesh,
        scratch_types=[
            pltpu.VMEM((RPT, H), jnp.int32),
            pltpu.VMEM((H, 16), jnp.float32),
            pltpu.VMEM((16, 16), jnp.float32),
            pltpu.VMEM_SHARED((NPAD, 16), jnp.float32),
        ],
    )
    edge_k = pl.kernel(
        _sc_edge_body,
        out_type=jax.ShapeDtypeStruct((NC * N, H), jnp.float32),
        mesh=mesh,
        scratch_types=[
            pltpu.VMEM((RPT, H), jnp.int32),
            pltpu.VMEM((RPT, H), jnp.int32),
            pltpu.VMEM((H, H), jnp.float32),
            pltpu.VMEM((16, H), jnp.float32),
            pltpu.SemaphoreType.DMA,
            pltpu.VMEM_SHARED((NPAD, H), jnp.float32),
        ],
    )
    return deg_k, edge_k


def _kwta(xw):
    bits = lax.bitcast_convert_type(xw, jnp.int32)
    skey = bits ^ (lax.shift_right_arithmetic(bits, 31) & jnp.int32(0x7FFFFFFF))
    minint = jnp.int32(-(2**31))
    prefix = jnp.zeros((xw.shape[0], 1), jnp.int32)
    for b in range(31, -1, -1):
        bit = minint if b == 31 else jnp.int32(1 << b)
        cand_u = prefix | bit
        cand_s = cand_u ^ minint
        cnt = jnp.sum((skey >= cand_s).astype(jnp.int32), axis=1, keepdims=True)
        prefix = jnp.where(cnt >= K, cand_u, prefix)
    kth_s = prefix ^ minint
    return jnp.where(skey >= kth_s, xw, jnp.float32(0.0))


BR = 1000


def _tc1_body(x_ref, w_ref, b_ref, o_ref):
    xw = jnp.dot(x_ref[...], w_ref[...], preferred_element_type=jnp.float32)
    o_ref[...] = _kwta(xw + b_ref[...])


def _tc2_body(h_ref, w_ref, d0_ref, d1_ref, o_ref):
    deg = 1.0 + d0_ref[...][:, 0:1] + d1_ref[...][:, 0:1]
    dinv = lax.rsqrt(deg)
    xw = jnp.dot(h_ref[...], w_ref[...], preferred_element_type=jnp.float32)
    o_ref[...] = xw * dinv


def _tc3_body(a0_ref, a1_ref, y_ref, d0_ref, d1_ref, b_ref, w_ref, o_ref):
    deg = 1.0 + d0_ref[...][:, 0:1] + d1_ref[...][:, 0:1]
    dinv = lax.rsqrt(deg)
    ssum = (a0_ref[...] + a1_ref[...] + y_ref[...]) * dinv + b_ref[...]
    h = _kwta(ssum)
    o_ref[...] = jnp.dot(h, w_ref[...], preferred_element_type=jnp.float32) * dinv


def _tc4_body(a0_ref, a1_ref, y_ref, d0_ref, d1_ref, b_ref, w_ref, b2_ref,
              o_ref):
    deg = 1.0 + d0_ref[...][:, 0:1] + d1_ref[...][:, 0:1]
    dinv = lax.rsqrt(deg)
    ssum = (a0_ref[...] + a1_ref[...] + y_ref[...]) * dinv + b_ref[...]
    h = _kwta(ssum)
    logits = jnp.dot(h, w_ref[...], preferred_element_type=jnp.float32)
    logits = logits + b2_ref[...]
    m = jnp.max(logits, axis=1, keepdims=True)
    lse = jnp.log(jnp.sum(jnp.exp(logits - m), axis=1, keepdims=True)) + m
    o_ref[...] = logits - lse


def _row_spec(width):
    return pl.BlockSpec((BR, width), lambda i: (i, 0))


def _full_spec(shape):
    return pl.BlockSpec(shape, lambda i: tuple(0 for _ in shape))


def kernel(x, edge_index, edge_weight, W1, b1, Wc0, bc0, Wc1, bc1, W2, b2):
    del edge_weight
    npad = EP - E
    pad = jnp.stack([jnp.zeros((npad,), edge_index.dtype),
                     jnp.full((npad,), N, edge_index.dtype)])
    ei3 = jnp.concatenate([edge_index, pad], axis=1).reshape(2, ER, H)
    b1r = b1.reshape(1, H)
    bc0r = bc0.reshape(1, H)
    bc1r = bc1.reshape(1, H)
    b2r = b2.reshape(1, C)

    grid = (N // BR,)
    sc_degree, sc_edge_pass = _build_sc_kernels()

    h0 = pl.pallas_call(
        _tc1_body,
        grid=grid,
        in_specs=[_row_spec(DIN), _full_spec((DIN, H)), _full_spec((1, H))],
        out_specs=_row_spec(H),
        out_shape=jax.ShapeDtypeStruct((N, H), jnp.float32),
    )(x, W1, b1r)

    degp = sc_degree(ei3)
    d0, d1 = degp[:N], degp[N:]

    y0 = pl.pallas_call(
        _tc2_body,
        grid=grid,
        in_specs=[_row_spec(H), _full_spec((H, H)), _row_spec(16),
                  _row_spec(16)],
        out_specs=_row_spec(H),
        out_shape=jax.ShapeDtypeStruct((N, H), jnp.float32),
    )(h0, Wc0, d0, d1)

    accp0 = sc_edge_pass(ei3, y0)
    a00, a01 = accp0[:N], accp0[N:]

    y1 = pl.pallas_call(
        _tc3_body,
        grid=grid,
        in_specs=[_row_spec(H), _row_spec(H), _row_spec(H), _row_spec(16),
                  _row_spec(16), _full_spec((1, H)), _full_spec((H, H))],
        out_specs=_row_spec(H),
        out_shape=jax.ShapeDtypeStruct((N, H), jnp.float32),
    )(a00, a01, y0, d0, d1, bc0r, Wc1)

    accp1 = sc_edge_pass(ei3, y1)
    a10, a11 = accp1[:N], accp1[N:]

    out = pl.pallas_call(
        _tc4_body,
        grid=grid,
        in_specs=[_row_spec(H), _row_spec(H), _row_spec(H), _row_spec(16),
                  _row_spec(16), _full_spec((1, H)), _full_spec((H, C)),
                  _full_spec((1, C))],
        out_specs=_row_spec(C),
        out_shape=jax.ShapeDtypeStruct((N, C), jnp.float32),
    )(a10, a11, y1, d0, d1, bc1r, W2, b2r)

    return out
